# Initial kernel scaffold; baseline (speedup 1.0000x reference)
import jax
import jax.numpy as jnp
from jax.experimental import pallas as pl


def kernel(x, edge_index, W1, b1, W2, b2):
    raise NotImplementedError("write your pallas kernel here")


# speedup vs baseline: 18.1374x; 18.1374x over previous
import jax
import jax.numpy as jnp
from jax import lax
from jax.experimental import pallas as pl
from jax.experimental.pallas import tpu as pltpu
from jax.experimental.pallas import tpu_sc as plsc

N_NODES = 10000
NPAD = 10240
N_EDGES = 320000
NW = 32
EDGES_PER_W = N_EDGES // NW
CHUNK = 80
NCHUNK = EDGES_PER_W // CHUNK
ROWS_PER_TILE = NPAD // 16
FP = 128
BLK = 640
NBLK = NPAD // BLK


def _deg_body(dst3, zeros_n, out, dst_v, ones_v, acc, sem):
    cid = lax.axis_index("c")
    sid = lax.axis_index("s")
    wid = sid * 2 + cid

    for i in range(CHUNK // 16):
        ones_v[pl.ds(i * 16, 16)] = jnp.ones((16,), jnp.float32)

    pltpu.sync_copy(zeros_n.at[pl.ds(sid * ROWS_PER_TILE, ROWS_PER_TILE)],
                    acc.at[pl.ds(sid * ROWS_PER_TILE, ROWS_PER_TILE)])
    pltpu.sync_copy(dst3.at[wid], dst_v)
    plsc.subcore_barrier()

    def chunk(c, carry):
        pltpu.sync_copy(ones_v, acc.at[dst_v.at[c]], add=True)
        return carry

    lax.fori_loop(0, NCHUNK, chunk, 0)
    plsc.subcore_barrier()

    pltpu.sync_copy(acc.at[pl.ds(sid * ROWS_PER_TILE, ROWS_PER_TILE)],
                    out.at[cid].at[pl.ds(sid * ROWS_PER_TILE, ROWS_PER_TILE)])


def _make_deg_kernel():
    return pl.kernel(
        _deg_body,
        out_type=jax.ShapeDtypeStruct((2, NPAD), jnp.float32),
        mesh=plsc.VectorSubcoreMesh(core_axis_name="c", subcore_axis_name="s"),
        scratch_types=[
            pltpu.VMEM((NCHUNK, CHUNK), jnp.int32),
            pltpu.VMEM((CHUNK,), jnp.float32),
            pltpu.VMEM_SHARED((NPAD,), jnp.float32),
            pltpu.SemaphoreType.DMA,
        ],
    )


def _agg_body(src3, dst3, g, zeros, out, src_v, dst_v, rows, acc, sem):
    cid = lax.axis_index("c")
    sid = lax.axis_index("s")
    wid = sid * 2 + cid
    rpt = ROWS_PER_TILE

    pltpu.sync_copy(zeros.at[pl.ds(sid * rpt, rpt)],
                    acc.at[pl.ds(sid * rpt, rpt)])
    pltpu.sync_copy(src3.at[wid], src_v)
    pltpu.sync_copy(dst3.at[wid], dst_v)
    plsc.subcore_barrier()

    def chunk(c, carry):
        pltpu.async_copy(g.at[src_v.at[c]], rows, sem).wait()
        pltpu.sync_copy(rows, acc.at[dst_v.at[c]], add=True)
        return carry

    lax.fori_loop(0, NCHUNK, chunk, 0)
    plsc.subcore_barrier()

    pltpu.sync_copy(acc.at[pl.ds(sid * rpt, rpt)],
                    out.at[cid].at[pl.ds(sid * rpt, rpt)])


def _make_agg_kernel():
    return pl.kernel(
        _agg_body,
        out_type=jax.ShapeDtypeStruct((2, NPAD, FP), jnp.float32),
        mesh=plsc.VectorSubcoreMesh(core_axis_name="c", subcore_axis_name="s"),
        scratch_types=[
            pltpu.VMEM((NCHUNK, CHUNK), jnp.int32),
            pltpu.VMEM((NCHUNK, CHUNK), jnp.int32),
            pltpu.VMEM((CHUNK, FP), jnp.float32),
            pltpu.VMEM_SHARED((NPAD, FP), jnp.float32),
            pltpu.SemaphoreType.DMA,
        ],
    )


def _tc_a_body(x_ref, w1_ref, d0_ref, d1_ref, g1_ref, dinv_ref):
    dinv = lax.rsqrt(d0_ref[...] + d1_ref[...] + 1.0)
    h = jnp.dot(x_ref[...], w1_ref[...], preferred_element_type=jnp.float32)
    g1_ref[...] = jnp.concatenate(
        [h * dinv, jnp.zeros((BLK, FP - 32), jnp.float32)], axis=1)
    dinv_ref[...] = dinv


def _tc_a(x, w1, d0, d1):
    return pl.pallas_call(
        _tc_a_body,
        grid=(NBLK,),
        in_specs=[
            pl.BlockSpec((BLK, 128), lambda i: (i, 0)),
            pl.BlockSpec((128, 32), lambda i: (0, 0)),
            pl.BlockSpec((BLK, 1), lambda i: (i, 0)),
            pl.BlockSpec((BLK, 1), lambda i: (i, 0)),
        ],
        out_specs=[
            pl.BlockSpec((BLK, FP), lambda i: (i, 0)),
            pl.BlockSpec((BLK, 1), lambda i: (i, 0)),
        ],
        out_shape=[
            jax.ShapeDtypeStruct((NPAD, FP), jnp.float32),
            jax.ShapeDtypeStruct((NPAD, 1), jnp.float32),
        ],
    )(x, w1, d0, d1)


def _tc_b_body(a0_ref, a1_ref, g1_ref, dinv_ref, b1_ref, w2_ref, g2_ref):
    dinv = dinv_ref[...]
    o1 = ((a0_ref[...] + a1_ref[...] + g1_ref[...])[:, :32] * dinv
          + b1_ref[...])
    o1 = jnp.maximum(o1, 0.0)
    h2 = jnp.dot(o1, w2_ref[...], preferred_element_type=jnp.float32)
    g2_ref[...] = jnp.concatenate(
        [h2 * dinv, jnp.zeros((BLK, FP - 64), jnp.float32)], axis=1)


def _tc_b(a0, a1, g1, dinv, b1, w2):
    return pl.pallas_call(
        _tc_b_body,
        grid=(NBLK,),
        in_specs=[
            pl.BlockSpec((BLK, FP), lambda i: (i, 0)),
            pl.BlockSpec((BLK, FP), lambda i: (i, 0)),
            pl.BlockSpec((BLK, FP), lambda i: (i, 0)),
            pl.BlockSpec((BLK, 1), lambda i: (i, 0)),
            pl.BlockSpec((1, 32), lambda i: (0, 0)),
            pl.BlockSpec((32, 64), lambda i: (0, 0)),
        ],
        out_specs=pl.BlockSpec((BLK, FP), lambda i: (i, 0)),
        out_shape=jax.ShapeDtypeStruct((NPAD, FP), jnp.float32),
    )(a0, a1, g1, dinv, b1, w2)


def _tc_c_body(a0_ref, a1_ref, g2_ref, dinv_ref, b2_ref, out_ref):
    out_ref[...] = ((a0_ref[...] + a1_ref[...] + g2_ref[...])[:, :64]
                    * dinv_ref[...] + b2_ref[...])


def _tc_c(a0, a1, g2, dinv, b2):
    return pl.pallas_call(
        _tc_c_body,
        grid=(NBLK,),
        in_specs=[
            pl.BlockSpec((BLK, FP), lambda i: (i, 0)),
            pl.BlockSpec((BLK, FP), lambda i: (i, 0)),
            pl.BlockSpec((BLK, FP), lambda i: (i, 0)),
            pl.BlockSpec((BLK, 1), lambda i: (i, 0)),
            pl.BlockSpec((1, 64), lambda i: (0, 0)),
        ],
        out_specs=pl.BlockSpec((BLK, 64), lambda i: (i, 0)),
        out_shape=jax.ShapeDtypeStruct((NPAD, 64), jnp.float32),
    )(a0, a1, g2, dinv, b2)


@jax.jit
def kernel(x, edge_index, W1, b1, W2, b2):
    src3 = edge_index[0].astype(jnp.int32).reshape(NW, NCHUNK, CHUNK)
    dst3 = edge_index[1].astype(jnp.int32).reshape(NW, NCHUNK, CHUNK)

    xp = jnp.zeros((NPAD, 128), jnp.float32).at[:N_NODES].set(x)
    zeros_n = jnp.zeros((NPAD,), jnp.float32)
    zeros2d = jnp.zeros((NPAD, FP), jnp.float32)

    deg = _make_deg_kernel()(dst3, zeros_n)
    d0 = deg[0].reshape(NPAD, 1)
    d1 = deg[1].reshape(NPAD, 1)

    g1, dinv = _tc_a(xp, W1, d0, d1)

    agg = _make_agg_kernel()
    agg1 = agg(src3, dst3, g1, zeros2d)
    g2 = _tc_b(agg1[0], agg1[1], g1, dinv, b1.reshape(1, 32), W2)

    agg2 = agg(src3, dst3, g2, zeros2d)
    out = _tc_c(agg2[0], agg2[1], g2, dinv, b2.reshape(1, 64))
    return out[:N_NODES]


# speedup vs baseline: 27.0713x; 1.4926x over previous
import jax
import jax.numpy as jnp
from jax import lax
from jax.experimental import pallas as pl
from jax.experimental.pallas import tpu as pltpu
from jax.experimental.pallas import tpu_sc as plsc

N_NODES = 10000
NPAD = 10240
N_EDGES = 320000
NW = 32
EDGES_PER_W = N_EDGES // NW
CHUNK = 80
NCHUNK = EDGES_PER_W // CHUNK
ROWS_PER_TILE = NPAD // 16
FP = 128
BLK = 640
NBLK = NPAD // BLK


def _deg_body(dst3, zeros_n, out, dst_v, ones_v, acc, sem):
    cid = lax.axis_index("c")
    sid = lax.axis_index("s")
    wid = sid * 2 + cid

    for i in range(CHUNK // 16):
        ones_v[pl.ds(i * 16, 16)] = jnp.ones((16,), jnp.float32)

    pltpu.sync_copy(zeros_n.at[pl.ds(sid * ROWS_PER_TILE, ROWS_PER_TILE)],
                    acc.at[pl.ds(sid * ROWS_PER_TILE, ROWS_PER_TILE)])
    pltpu.sync_copy(dst3.at[wid], dst_v)
    plsc.subcore_barrier()

    def chunk(c, carry):
        pltpu.sync_copy(ones_v, acc.at[dst_v.at[c]], add=True)
        return carry

    lax.fori_loop(0, NCHUNK, chunk, 0)
    plsc.subcore_barrier()

    pltpu.sync_copy(acc.at[pl.ds(sid * ROWS_PER_TILE, ROWS_PER_TILE)],
                    out.at[cid].at[pl.ds(sid * ROWS_PER_TILE, ROWS_PER_TILE)])


def _make_deg_kernel():
    return pl.kernel(
        _deg_body,
        out_type=jax.ShapeDtypeStruct((2, NPAD), jnp.float32),
        mesh=plsc.VectorSubcoreMesh(core_axis_name="c", subcore_axis_name="s"),
        scratch_types=[
            pltpu.VMEM((NCHUNK, CHUNK), jnp.int32),
            pltpu.VMEM((CHUNK,), jnp.float32),
            pltpu.VMEM_SHARED((NPAD,), jnp.float32),
            pltpu.SemaphoreType.DMA,
        ],
    )


def _agg_body(src3, dst4, g, zeros, out, src_v, db0, db1, rows0, rows1, acc,
              sem0, sem1, semd0, semd1):
    cid = lax.axis_index("c")
    sid = lax.axis_index("s")
    wid = sid * 2 + cid
    rpt = ROWS_PER_TILE
    dst2 = dst4.at[wid]

    pltpu.sync_copy(zeros.at[pl.ds(sid * rpt, rpt)],
                    acc.at[pl.ds(sid * rpt, rpt)])
    pltpu.sync_copy(src3.at[wid], src_v)
    plsc.subcore_barrier()

    pltpu.async_copy(g.at[src_v.at[0]], rows0, sem0)
    pltpu.async_copy(dst2.at[0], db0, semd0)

    def pair(i, carry):
        c0 = 2 * i
        pltpu.async_copy(g.at[src_v.at[c0 + 1]], rows1, sem1)
        pltpu.async_copy(dst2.at[c0 + 1], db1, semd1)
        pltpu.make_async_copy(g.at[src_v.at[c0]], rows0, sem0).wait()
        pltpu.make_async_copy(dst2.at[c0], db0, semd0).wait()
        pltpu.sync_copy(rows0, acc.at[db0.at[0]], add=True)
        pltpu.async_copy(g.at[src_v.at[c0 + 2]], rows0, sem0)
        pltpu.async_copy(dst2.at[c0 + 2], db0, semd0)
        pltpu.make_async_copy(g.at[src_v.at[c0 + 1]], rows1, sem1).wait()
        pltpu.make_async_copy(dst2.at[c0 + 1], db1, semd1).wait()
        pltpu.sync_copy(rows1, acc.at[db1.at[0]], add=True)
        return carry

    lax.fori_loop(0, (NCHUNK - 1) // 2, pair, 0)
    pltpu.make_async_copy(g.at[src_v.at[NCHUNK - 1]], rows0, sem0).wait()
    pltpu.make_async_copy(dst2.at[NCHUNK - 1], db0, semd0).wait()
    pltpu.sync_copy(rows0, acc.at[db0.at[0]], add=True)
    plsc.subcore_barrier()

    pltpu.sync_copy(acc.at[pl.ds(sid * rpt, rpt)],
                    out.at[cid].at[pl.ds(sid * rpt, rpt)])


def _make_agg_kernel():
    return pl.kernel(
        _agg_body,
        out_type=jax.ShapeDtypeStruct((2, NPAD, FP), jnp.float32),
        mesh=plsc.VectorSubcoreMesh(core_axis_name="c", subcore_axis_name="s"),
        scratch_types=[
            pltpu.VMEM((NCHUNK, CHUNK), jnp.int32),
            pltpu.VMEM((1, CHUNK), jnp.int32),
            pltpu.VMEM((1, CHUNK), jnp.int32),
            pltpu.VMEM((CHUNK, FP), jnp.float32),
            pltpu.VMEM((CHUNK, FP), jnp.float32),
            pltpu.VMEM_SHARED((NPAD, FP), jnp.float32),
            pltpu.SemaphoreType.DMA,
            pltpu.SemaphoreType.DMA,
            pltpu.SemaphoreType.DMA,
            pltpu.SemaphoreType.DMA,
        ],
    )


def _tc_a_body(x_ref, w1_ref, d0_ref, d1_ref, g1_ref, dinv_ref):
    dinv = lax.rsqrt(d0_ref[...] + d1_ref[...] + 1.0)
    h = jnp.dot(x_ref[...], w1_ref[...], preferred_element_type=jnp.float32)
    g1_ref[...] = jnp.concatenate(
        [h * dinv, jnp.zeros((BLK, FP - 32), jnp.float32)], axis=1)
    dinv_ref[...] = dinv


def _tc_a(x, w1, d0, d1):
    return pl.pallas_call(
        _tc_a_body,
        grid=(NBLK,),
        in_specs=[
            pl.BlockSpec((BLK, 128), lambda i: (i, 0)),
            pl.BlockSpec((128, 32), lambda i: (0, 0)),
            pl.BlockSpec((BLK, 1), lambda i: (i, 0)),
            pl.BlockSpec((BLK, 1), lambda i: (i, 0)),
        ],
        out_specs=[
            pl.BlockSpec((BLK, FP), lambda i: (i, 0)),
            pl.BlockSpec((BLK, 1), lambda i: (i, 0)),
        ],
        out_shape=[
            jax.ShapeDtypeStruct((NPAD, FP), jnp.float32),
            jax.ShapeDtypeStruct((NPAD, 1), jnp.float32),
        ],
    )(x, w1, d0, d1)


def _tc_b_body(a0_ref, a1_ref, g1_ref, dinv_ref, b1_ref, w2_ref, g2_ref):
    dinv = dinv_ref[...]
    o1 = ((a0_ref[...] + a1_ref[...] + g1_ref[...])[:, :32] * dinv
          + b1_ref[...])
    o1 = jnp.maximum(o1, 0.0)
    h2 = jnp.dot(o1, w2_ref[...], preferred_element_type=jnp.float32)
    g2_ref[...] = jnp.concatenate(
        [h2 * dinv, jnp.zeros((BLK, FP - 64), jnp.float32)], axis=1)


def _tc_b(a0, a1, g1, dinv, b1, w2):
    return pl.pallas_call(
        _tc_b_body,
        grid=(NBLK,),
        in_specs=[
            pl.BlockSpec((BLK, FP), lambda i: (i, 0)),
            pl.BlockSpec((BLK, FP), lambda i: (i, 0)),
            pl.BlockSpec((BLK, FP), lambda i: (i, 0)),
            pl.BlockSpec((BLK, 1), lambda i: (i, 0)),
            pl.BlockSpec((1, 32), lambda i: (0, 0)),
            pl.BlockSpec((32, 64), lambda i: (0, 0)),
        ],
        out_specs=pl.BlockSpec((BLK, FP), lambda i: (i, 0)),
        out_shape=jax.ShapeDtypeStruct((NPAD, FP), jnp.float32),
    )(a0, a1, g1, dinv, b1, w2)


def _tc_c_body(a0_ref, a1_ref, g2_ref, dinv_ref, b2_ref, out_ref):
    out_ref[...] = ((a0_ref[...] + a1_ref[...] + g2_ref[...])[:, :64]
                    * dinv_ref[...] + b2_ref[...])


def _tc_c(a0, a1, g2, dinv, b2):
    return pl.pallas_call(
        _tc_c_body,
        grid=(NBLK,),
        in_specs=[
            pl.BlockSpec((BLK, FP), lambda i: (i, 0)),
            pl.BlockSpec((BLK, FP), lambda i: (i, 0)),
            pl.BlockSpec((BLK, FP), lambda i: (i, 0)),
            pl.BlockSpec((BLK, 1), lambda i: (i, 0)),
            pl.BlockSpec((1, 64), lambda i: (0, 0)),
        ],
        out_specs=pl.BlockSpec((BLK, 64), lambda i: (i, 0)),
        out_shape=jax.ShapeDtypeStruct((NPAD, 64), jnp.float32),
    )(a0, a1, g2, dinv, b2)


@jax.jit
def kernel(x, edge_index, W1, b1, W2, b2):
    src3 = edge_index[0].astype(jnp.int32).reshape(NW, NCHUNK, CHUNK)
    dst_i32 = edge_index[1].astype(jnp.int32)
    dst3 = dst_i32.reshape(NW, NCHUNK, CHUNK)
    dst4 = dst_i32.reshape(NW, NCHUNK, 1, CHUNK)

    xp = jnp.zeros((NPAD, 128), jnp.float32).at[:N_NODES].set(x)
    zeros_n = jnp.zeros((NPAD,), jnp.float32)
    zeros2d = jnp.zeros((NPAD, FP), jnp.float32)

    deg = _make_deg_kernel()(dst3, zeros_n)
    d0 = deg[0].reshape(NPAD, 1)
    d1 = deg[1].reshape(NPAD, 1)

    g1, dinv = _tc_a(xp, W1, d0, d1)

    agg = _make_agg_kernel()
    agg1 = agg(src3, dst4, g1, zeros2d)
    g2 = _tc_b(agg1[0], agg1[1], g1, dinv, b1.reshape(1, 32), W2)

    agg2 = agg(src3, dst4, g2, zeros2d)
    out = _tc_c(agg2[0], agg2[1], g2, dinv, b2.reshape(1, 64))
    return out[:N_NODES]


# speedup vs baseline: 27.9244x; 1.0315x over previous
import jax
import jax.numpy as jnp
from jax import lax
from jax.experimental import pallas as pl
from jax.experimental.pallas import tpu as pltpu
from jax.experimental.pallas import tpu_sc as plsc

N_NODES = 10000
NPAD = 10240
N_EDGES = 320000
NW = 32
EDGES_PER_W = N_EDGES // NW
CHUNK = 80
NCHUNK = EDGES_PER_W // CHUNK
ROWS_PER_TILE = NPAD // 16
FP = 128
BLK = 1000
NBLK = N_NODES // BLK


def _deg_body(dst3, zeros_n, out, dst_v, ones_v, acc, sem):
    cid = lax.axis_index("c")
    sid = lax.axis_index("s")
    wid = sid * 2 + cid

    for i in range(CHUNK // 16):
        ones_v[pl.ds(i * 16, 16)] = jnp.ones((16,), jnp.float32)

    pltpu.sync_copy(zeros_n.at[pl.ds(sid * ROWS_PER_TILE, ROWS_PER_TILE)],
                    acc.at[pl.ds(sid * ROWS_PER_TILE, ROWS_PER_TILE)])
    pltpu.sync_copy(dst3.at[wid], dst_v)
    plsc.subcore_barrier()

    def chunk(c, carry):
        pltpu.sync_copy(ones_v, acc.at[dst_v.at[c]], add=True)
        return carry

    lax.fori_loop(0, NCHUNK, chunk, 0)
    plsc.subcore_barrier()

    pltpu.sync_copy(acc.at[pl.ds(sid * ROWS_PER_TILE, ROWS_PER_TILE)],
                    out.at[cid].at[pl.ds(sid * ROWS_PER_TILE, ROWS_PER_TILE)])


def _make_deg_kernel():
    return pl.kernel(
        _deg_body,
        out_type=jax.ShapeDtypeStruct((2, NPAD), jnp.float32),
        mesh=plsc.VectorSubcoreMesh(core_axis_name="c", subcore_axis_name="s"),
        scratch_types=[
            pltpu.VMEM((NCHUNK, CHUNK), jnp.int32),
            pltpu.VMEM((CHUNK,), jnp.float32),
            pltpu.VMEM_SHARED((NPAD,), jnp.float32),
            pltpu.SemaphoreType.DMA,
        ],
    )


def _agg_body(src3, dst4, g, zeros, out, src_v, db0, db1, rows0, rows1, acc,
              sem0, sem1, semd0, semd1):
    cid = lax.axis_index("c")
    sid = lax.axis_index("s")
    wid = sid * 2 + cid
    rpt = ROWS_PER_TILE
    dst2 = dst4.at[wid]

    pltpu.sync_copy(zeros.at[pl.ds(sid * rpt, rpt)],
                    acc.at[pl.ds(sid * rpt, rpt)])
    pltpu.sync_copy(src3.at[wid], src_v)
    plsc.subcore_barrier()

    pltpu.async_copy(g.at[src_v.at[0]], rows0, sem0)
    pltpu.async_copy(dst2.at[0], db0, semd0)

    def pair(i, carry):
        c0 = 2 * i
        pltpu.async_copy(g.at[src_v.at[c0 + 1]], rows1, sem1)
        pltpu.async_copy(dst2.at[c0 + 1], db1, semd1)
        pltpu.make_async_copy(g.at[src_v.at[c0]], rows0, sem0).wait()
        pltpu.make_async_copy(dst2.at[c0], db0, semd0).wait()
        pltpu.sync_copy(rows0, acc.at[db0.at[0]], add=True)
        pltpu.async_copy(g.at[src_v.at[c0 + 2]], rows0, sem0)
        pltpu.async_copy(dst2.at[c0 + 2], db0, semd0)
        pltpu.make_async_copy(g.at[src_v.at[c0 + 1]], rows1, sem1).wait()
        pltpu.make_async_copy(dst2.at[c0 + 1], db1, semd1).wait()
        pltpu.sync_copy(rows1, acc.at[db1.at[0]], add=True)
        return carry

    lax.fori_loop(0, (NCHUNK - 1) // 2, pair, 0)
    pltpu.make_async_copy(g.at[src_v.at[NCHUNK - 1]], rows0, sem0).wait()
    pltpu.make_async_copy(dst2.at[NCHUNK - 1], db0, semd0).wait()
    pltpu.sync_copy(rows0, acc.at[db0.at[0]], add=True)
    plsc.subcore_barrier()

    pltpu.sync_copy(acc.at[pl.ds(sid * rpt, rpt)],
                    out.at[cid].at[pl.ds(sid * rpt, rpt)])


def _make_agg_kernel():
    return pl.kernel(
        _agg_body,
        out_type=jax.ShapeDtypeStruct((2, NPAD, FP), jnp.float32),
        mesh=plsc.VectorSubcoreMesh(core_axis_name="c", subcore_axis_name="s"),
        scratch_types=[
            pltpu.VMEM((NCHUNK, CHUNK), jnp.int32),
            pltpu.VMEM((1, CHUNK), jnp.int32),
            pltpu.VMEM((1, CHUNK), jnp.int32),
            pltpu.VMEM((CHUNK, FP), jnp.float32),
            pltpu.VMEM((CHUNK, FP), jnp.float32),
            pltpu.VMEM_SHARED((NPAD, FP), jnp.float32),
            pltpu.SemaphoreType.DMA,
            pltpu.SemaphoreType.DMA,
            pltpu.SemaphoreType.DMA,
            pltpu.SemaphoreType.DMA,
        ],
    )


def _tc_a_body(x_ref, w1_ref, d0_ref, d1_ref, g1_ref, dinv_ref):
    dinv = lax.rsqrt(d0_ref[...] + d1_ref[...] + 1.0)
    h = jnp.dot(x_ref[...], w1_ref[...], preferred_element_type=jnp.float32)
    g1_ref[...] = jnp.concatenate(
        [h * dinv, jnp.zeros((BLK, FP - 32), jnp.float32)], axis=1)
    dinv_ref[...] = dinv


def _tc_a(x, w1, d0, d1):
    return pl.pallas_call(
        _tc_a_body,
        grid=(NBLK,),
        in_specs=[
            pl.BlockSpec((BLK, 128), lambda i: (i, 0)),
            pl.BlockSpec((128, 32), lambda i: (0, 0)),
            pl.BlockSpec((BLK, 1), lambda i: (i, 0)),
            pl.BlockSpec((BLK, 1), lambda i: (i, 0)),
        ],
        out_specs=[
            pl.BlockSpec((BLK, FP), lambda i: (i, 0)),
            pl.BlockSpec((BLK, 1), lambda i: (i, 0)),
        ],
        out_shape=[
            jax.ShapeDtypeStruct((N_NODES, FP), jnp.float32),
            jax.ShapeDtypeStruct((N_NODES, 1), jnp.float32),
        ],
    )(x, w1, d0, d1)


def _tc_b_body(a0_ref, a1_ref, g1_ref, dinv_ref, b1_ref, w2_ref, g2_ref):
    dinv = dinv_ref[...]
    o1 = ((a0_ref[...] + a1_ref[...] + g1_ref[...])[:, :32] * dinv
          + b1_ref[...])
    o1 = jnp.maximum(o1, 0.0)
    h2 = jnp.dot(o1, w2_ref[...], preferred_element_type=jnp.float32)
    g2_ref[...] = jnp.concatenate(
        [h2 * dinv, jnp.zeros((BLK, FP - 64), jnp.float32)], axis=1)


def _tc_b(a0, a1, g1, dinv, b1, w2):
    return pl.pallas_call(
        _tc_b_body,
        grid=(NBLK,),
        in_specs=[
            pl.BlockSpec((BLK, FP), lambda i: (i, 0)),
            pl.BlockSpec((BLK, FP), lambda i: (i, 0)),
            pl.BlockSpec((BLK, FP), lambda i: (i, 0)),
            pl.BlockSpec((BLK, 1), lambda i: (i, 0)),
            pl.BlockSpec((1, 32), lambda i: (0, 0)),
            pl.BlockSpec((32, 64), lambda i: (0, 0)),
        ],
        out_specs=pl.BlockSpec((BLK, FP), lambda i: (i, 0)),
        out_shape=jax.ShapeDtypeStruct((N_NODES, FP), jnp.float32),
    )(a0, a1, g1, dinv, b1, w2)


def _tc_c_body(a0_ref, a1_ref, g2_ref, dinv_ref, b2_ref, out_ref):
    out_ref[...] = ((a0_ref[...] + a1_ref[...] + g2_ref[...])[:, :64]
                    * dinv_ref[...] + b2_ref[...])


def _tc_c(a0, a1, g2, dinv, b2):
    return pl.pallas_call(
        _tc_c_body,
        grid=(NBLK,),
        in_specs=[
            pl.BlockSpec((BLK, FP), lambda i: (i, 0)),
            pl.BlockSpec((BLK, FP), lambda i: (i, 0)),
            pl.BlockSpec((BLK, FP), lambda i: (i, 0)),
            pl.BlockSpec((BLK, 1), lambda i: (i, 0)),
            pl.BlockSpec((1, 64), lambda i: (0, 0)),
        ],
        out_specs=pl.BlockSpec((BLK, 64), lambda i: (i, 0)),
        out_shape=jax.ShapeDtypeStruct((N_NODES, 64), jnp.float32),
    )(a0, a1, g2, dinv, b2)


@jax.jit
def kernel(x, edge_index, W1, b1, W2, b2):
    src3 = edge_index[0].astype(jnp.int32).reshape(NW, NCHUNK, CHUNK)
    dst_i32 = edge_index[1].astype(jnp.int32)
    dst3 = dst_i32.reshape(NW, NCHUNK, CHUNK)
    dst4 = dst_i32.reshape(NW, NCHUNK, 1, CHUNK)

    zeros_n = jnp.zeros((NPAD,), jnp.float32)
    zeros2d = jnp.zeros((NPAD, FP), jnp.float32)

    deg = _make_deg_kernel()(dst3, zeros_n)
    d0 = deg[0, :N_NODES].reshape(N_NODES, 1)
    d1 = deg[1, :N_NODES].reshape(N_NODES, 1)

    g1, dinv = _tc_a(x, W1, d0, d1)

    agg = _make_agg_kernel()
    agg1 = agg(src3, dst4, g1, zeros2d)
    g2 = _tc_b(agg1[0], agg1[1], g1, dinv, b1.reshape(1, 32), W2)

    agg2 = agg(src3, dst4, g2, zeros2d)
    return _tc_c(agg2[0], agg2[1], g2, dinv, b2.reshape(1, 64))


# speedup vs baseline: 34.5845x; 1.2385x over previous
import jax
import jax.numpy as jnp
from jax import lax
from jax.experimental import pallas as pl
from jax.experimental.pallas import tpu as pltpu
from jax.experimental.pallas import tpu_sc as plsc

N_NODES = 10000
NPAD = 10240
N_EDGES = 320000
NW = 32
EDGES_PER_W = N_EDGES // NW
CHUNK = 80
NCHUNK = EDGES_PER_W // CHUNK
ROWS_PER_TILE = NPAD // 16
BLK = 1000
NBLK = N_NODES // BLK


def _deg_body(dst3, zeros_n, out, dst_v, ones_v, acc, sem):
    cid = lax.axis_index("c")
    sid = lax.axis_index("s")
    wid = sid * 2 + cid

    for i in range(CHUNK // 16):
        ones_v[pl.ds(i * 16, 16)] = jnp.ones((16,), jnp.float32)

    pltpu.sync_copy(zeros_n.at[pl.ds(sid * ROWS_PER_TILE, ROWS_PER_TILE)],
                    acc.at[pl.ds(sid * ROWS_PER_TILE, ROWS_PER_TILE)])
    pltpu.sync_copy(dst3.at[wid], dst_v)
    plsc.subcore_barrier()

    def chunk(c, carry):
        pltpu.sync_copy(ones_v, acc.at[dst_v.at[c]], add=True)
        return carry

    lax.fori_loop(0, NCHUNK, chunk, 0)
    plsc.subcore_barrier()

    pltpu.sync_copy(acc.at[pl.ds(sid * ROWS_PER_TILE, ROWS_PER_TILE)],
                    out.at[cid].at[pl.ds(sid * ROWS_PER_TILE, ROWS_PER_TILE)])


def _make_deg_kernel():
    return pl.kernel(
        _deg_body,
        out_type=jax.ShapeDtypeStruct((2, NPAD), jnp.float32),
        mesh=plsc.VectorSubcoreMesh(core_axis_name="c", subcore_axis_name="s"),
        compiler_params=pltpu.CompilerParams(use_tc_tiling_on_sc=False),
        scratch_types=[
            pltpu.VMEM((NCHUNK, CHUNK), jnp.int32),
            pltpu.VMEM((CHUNK,), jnp.float32),
            pltpu.VMEM_SHARED((NPAD,), jnp.float32),
            pltpu.SemaphoreType.DMA,
        ],
    )


def _agg_body(src3, dst4, g, zeros, out, src_v, db0, db1, rows0, rows1, acc,
              sem0, sem1, semd0, semd1):
    cid = lax.axis_index("c")
    sid = lax.axis_index("s")
    wid = sid * 2 + cid
    rpt = ROWS_PER_TILE
    dst2 = dst4.at[wid]

    pltpu.sync_copy(zeros.at[pl.ds(sid * rpt, rpt)],
                    acc.at[pl.ds(sid * rpt, rpt)])
    pltpu.sync_copy(src3.at[wid], src_v)
    plsc.subcore_barrier()

    pltpu.async_copy(g.at[src_v.at[0]], rows0, sem0)
    pltpu.async_copy(dst2.at[0], db0, semd0)

    def pair(i, carry):
        c0 = 2 * i
        pltpu.async_copy(g.at[src_v.at[c0 + 1]], rows1, sem1)
        pltpu.async_copy(dst2.at[c0 + 1], db1, semd1)
        pltpu.make_async_copy(g.at[src_v.at[c0]], rows0, sem0).wait()
        pltpu.make_async_copy(dst2.at[c0], db0, semd0).wait()
        pltpu.sync_copy(rows0, acc.at[db0.at[0]], add=True)
        pltpu.async_copy(g.at[src_v.at[c0 + 2]], rows0, sem0)
        pltpu.async_copy(dst2.at[c0 + 2], db0, semd0)
        pltpu.make_async_copy(g.at[src_v.at[c0 + 1]], rows1, sem1).wait()
        pltpu.make_async_copy(dst2.at[c0 + 1], db1, semd1).wait()
        pltpu.sync_copy(rows1, acc.at[db1.at[0]], add=True)
        return carry

    lax.fori_loop(0, (NCHUNK - 1) // 2, pair, 0)
    pltpu.make_async_copy(g.at[src_v.at[NCHUNK - 1]], rows0, sem0).wait()
    pltpu.make_async_copy(dst2.at[NCHUNK - 1], db0, semd0).wait()
    pltpu.sync_copy(rows0, acc.at[db0.at[0]], add=True)
    plsc.subcore_barrier()

    pltpu.sync_copy(acc.at[pl.ds(sid * rpt, rpt)],
                    out.at[cid].at[pl.ds(sid * rpt, rpt)])


def _make_agg_kernel(feat):
    return pl.kernel(
        _agg_body,
        out_type=jax.ShapeDtypeStruct((2, NPAD, feat), jnp.float32),
        mesh=plsc.VectorSubcoreMesh(core_axis_name="c", subcore_axis_name="s"),
        compiler_params=pltpu.CompilerParams(use_tc_tiling_on_sc=False),
        scratch_types=[
            pltpu.VMEM((NCHUNK, CHUNK), jnp.int32),
            pltpu.VMEM((1, CHUNK), jnp.int32),
            pltpu.VMEM((1, CHUNK), jnp.int32),
            pltpu.VMEM((CHUNK, feat), jnp.float32),
            pltpu.VMEM((CHUNK, feat), jnp.float32),
            pltpu.VMEM_SHARED((NPAD, feat), jnp.float32),
            pltpu.SemaphoreType.DMA,
            pltpu.SemaphoreType.DMA,
            pltpu.SemaphoreType.DMA,
            pltpu.SemaphoreType.DMA,
        ],
    )


def _tc_a_body(x_ref, w1_ref, d0_ref, d1_ref, g1_ref, dinv_ref):
    dinv = lax.rsqrt(d0_ref[...] + d1_ref[...] + 1.0)
    h = jnp.dot(x_ref[...], w1_ref[...], preferred_element_type=jnp.float32)
    g1_ref[...] = h * dinv
    dinv_ref[...] = dinv


def _tc_a(x, w1, d0, d1):
    return pl.pallas_call(
        _tc_a_body,
        grid=(NBLK,),
        in_specs=[
            pl.BlockSpec((BLK, 128), lambda i: (i, 0)),
            pl.BlockSpec((128, 32), lambda i: (0, 0)),
            pl.BlockSpec((BLK, 1), lambda i: (i, 0)),
            pl.BlockSpec((BLK, 1), lambda i: (i, 0)),
        ],
        out_specs=[
            pl.BlockSpec((BLK, 32), lambda i: (i, 0)),
            pl.BlockSpec((BLK, 1), lambda i: (i, 0)),
        ],
        out_shape=[
            jax.ShapeDtypeStruct((N_NODES, 32), jnp.float32),
            jax.ShapeDtypeStruct((N_NODES, 1), jnp.float32),
        ],
    )(x, w1, d0, d1)


def _tc_b_body(a0_ref, a1_ref, g1_ref, dinv_ref, b1_ref, w2_ref, g2_ref):
    dinv = dinv_ref[...]
    o1 = ((a0_ref[...] + a1_ref[...] + g1_ref[...]) * dinv
          + b1_ref[...])
    o1 = jnp.maximum(o1, 0.0)
    h2 = jnp.dot(o1, w2_ref[...], preferred_element_type=jnp.float32)
    g2_ref[...] = h2 * dinv


def _tc_b(a0, a1, g1, dinv, b1, w2):
    return pl.pallas_call(
        _tc_b_body,
        grid=(NBLK,),
        in_specs=[
            pl.BlockSpec((BLK, 32), lambda i: (i, 0)),
            pl.BlockSpec((BLK, 32), lambda i: (i, 0)),
            pl.BlockSpec((BLK, 32), lambda i: (i, 0)),
            pl.BlockSpec((BLK, 1), lambda i: (i, 0)),
            pl.BlockSpec((1, 32), lambda i: (0, 0)),
            pl.BlockSpec((32, 64), lambda i: (0, 0)),
        ],
        out_specs=pl.BlockSpec((BLK, 64), lambda i: (i, 0)),
        out_shape=jax.ShapeDtypeStruct((N_NODES, 64), jnp.float32),
    )(a0, a1, g1, dinv, b1, w2)


def _tc_c_body(a0_ref, a1_ref, g2_ref, dinv_ref, b2_ref, out_ref):
    out_ref[...] = ((a0_ref[...] + a1_ref[...] + g2_ref[...])
                    * dinv_ref[...] + b2_ref[...])


def _tc_c(a0, a1, g2, dinv, b2):
    return pl.pallas_call(
        _tc_c_body,
        grid=(NBLK,),
        in_specs=[
            pl.BlockSpec((BLK, 64), lambda i: (i, 0)),
            pl.BlockSpec((BLK, 64), lambda i: (i, 0)),
            pl.BlockSpec((BLK, 64), lambda i: (i, 0)),
            pl.BlockSpec((BLK, 1), lambda i: (i, 0)),
            pl.BlockSpec((1, 64), lambda i: (0, 0)),
        ],
        out_specs=pl.BlockSpec((BLK, 64), lambda i: (i, 0)),
        out_shape=jax.ShapeDtypeStruct((N_NODES, 64), jnp.float32),
    )(a0, a1, g2, dinv, b2)


@jax.jit
def kernel(x, edge_index, W1, b1, W2, b2):
    src3 = edge_index[0].astype(jnp.int32).reshape(NW, NCHUNK, CHUNK)
    dst_i32 = edge_index[1].astype(jnp.int32)
    dst3 = dst_i32.reshape(NW, NCHUNK, CHUNK)
    dst4 = dst_i32.reshape(NW, NCHUNK, 1, CHUNK)

    zeros_n = jnp.zeros((NPAD,), jnp.float32)
    zeros32 = jnp.zeros((NPAD, 32), jnp.float32)
    zeros64 = jnp.zeros((NPAD, 64), jnp.float32)

    deg = _make_deg_kernel()(dst3, zeros_n)
    d0 = deg[0, :N_NODES].reshape(N_NODES, 1)
    d1 = deg[1, :N_NODES].reshape(N_NODES, 1)

    g1, dinv = _tc_a(x, W1, d0, d1)

    agg1 = _make_agg_kernel(32)(src3, dst4, g1, zeros32)
    g2 = _tc_b(agg1[0], agg1[1], g1, dinv, b1.reshape(1, 32), W2)

    agg2 = _make_agg_kernel(64)(src3, dst4, g2, zeros64)
    return _tc_c(agg2[0], agg2[1], g2, dinv, b2.reshape(1, 64))


# speedup vs baseline: 34.5978x; 1.0004x over previous
import jax
import jax.numpy as jnp
from jax import lax
from jax.experimental import pallas as pl
from jax.experimental.pallas import tpu as pltpu
from jax.experimental.pallas import tpu_sc as plsc

N_NODES = 10000
NPAD = 10240
N_EDGES = 320000
NW = 32
EDGES_PER_W = N_EDGES // NW
CHUNK = 80
NCHUNK = EDGES_PER_W // CHUNK
ROWS_PER_TILE = NPAD // 16
BLK = 1000
NBLK = N_NODES // BLK


def _deg_body(dst3, zeros_n, out, dst_v, ones_v, acc, sem):
    cid = lax.axis_index("c")
    sid = lax.axis_index("s")
    wid = sid * 2 + cid

    for i in range(CHUNK // 16):
        ones_v[pl.ds(i * 16, 16)] = jnp.ones((16,), jnp.float32)

    pltpu.sync_copy(zeros_n.at[pl.ds(sid * ROWS_PER_TILE, ROWS_PER_TILE)],
                    acc.at[pl.ds(sid * ROWS_PER_TILE, ROWS_PER_TILE)])
    pltpu.sync_copy(dst3.at[wid], dst_v)
    plsc.subcore_barrier()

    def chunk(c, carry):
        pltpu.sync_copy(ones_v, acc.at[dst_v.at[c]], add=True)
        return carry

    lax.fori_loop(0, NCHUNK, chunk, 0)
    plsc.subcore_barrier()

    pltpu.sync_copy(acc.at[pl.ds(sid * ROWS_PER_TILE, ROWS_PER_TILE)],
                    out.at[cid].at[pl.ds(sid * ROWS_PER_TILE, ROWS_PER_TILE)])


def _make_deg_kernel():
    return pl.kernel(
        _deg_body,
        out_type=jax.ShapeDtypeStruct((2, NPAD), jnp.float32),
        mesh=plsc.VectorSubcoreMesh(core_axis_name="c", subcore_axis_name="s"),
        compiler_params=pltpu.CompilerParams(use_tc_tiling_on_sc=False, skip_device_barrier=True),
        scratch_types=[
            pltpu.VMEM((NCHUNK, CHUNK), jnp.int32),
            pltpu.VMEM((CHUNK,), jnp.float32),
            pltpu.VMEM_SHARED((NPAD,), jnp.float32),
            pltpu.SemaphoreType.DMA,
        ],
    )


def _agg_body(src3, dst4, g, zeros, out, src_v, db0, db1, rows0, rows1, acc,
              sem0, sem1, semd0, semd1):
    cid = lax.axis_index("c")
    sid = lax.axis_index("s")
    wid = sid * 2 + cid
    rpt = ROWS_PER_TILE
    dst2 = dst4.at[wid]

    pltpu.sync_copy(zeros.at[pl.ds(sid * rpt, rpt)],
                    acc.at[pl.ds(sid * rpt, rpt)])
    pltpu.sync_copy(src3.at[wid], src_v)
    plsc.subcore_barrier()

    pltpu.async_copy(g.at[src_v.at[0]], rows0, sem0)
    pltpu.async_copy(dst2.at[0], db0, semd0)

    def pair(i, carry):
        c0 = 2 * i
        pltpu.async_copy(g.at[src_v.at[c0 + 1]], rows1, sem1)
        pltpu.async_copy(dst2.at[c0 + 1], db1, semd1)
        pltpu.make_async_copy(g.at[src_v.at[c0]], rows0, sem0).wait()
        pltpu.make_async_copy(dst2.at[c0], db0, semd0).wait()
        pltpu.sync_copy(rows0, acc.at[db0.at[0]], add=True)
        pltpu.async_copy(g.at[src_v.at[c0 + 2]], rows0, sem0)
        pltpu.async_copy(dst2.at[c0 + 2], db0, semd0)
        pltpu.make_async_copy(g.at[src_v.at[c0 + 1]], rows1, sem1).wait()
        pltpu.make_async_copy(dst2.at[c0 + 1], db1, semd1).wait()
        pltpu.sync_copy(rows1, acc.at[db1.at[0]], add=True)
        return carry

    lax.fori_loop(0, (NCHUNK - 1) // 2, pair, 0)
    pltpu.make_async_copy(g.at[src_v.at[NCHUNK - 1]], rows0, sem0).wait()
    pltpu.make_async_copy(dst2.at[NCHUNK - 1], db0, semd0).wait()
    pltpu.sync_copy(rows0, acc.at[db0.at[0]], add=True)
    plsc.subcore_barrier()

    pltpu.sync_copy(acc.at[pl.ds(sid * rpt, rpt)],
                    out.at[cid].at[pl.ds(sid * rpt, rpt)])


def _make_agg_kernel(feat):
    return pl.kernel(
        _agg_body,
        out_type=jax.ShapeDtypeStruct((2, NPAD, feat), jnp.float32),
        mesh=plsc.VectorSubcoreMesh(core_axis_name="c", subcore_axis_name="s"),
        compiler_params=pltpu.CompilerParams(use_tc_tiling_on_sc=False, skip_device_barrier=True),
        scratch_types=[
            pltpu.VMEM((NCHUNK, CHUNK), jnp.int32),
            pltpu.VMEM((1, CHUNK), jnp.int32),
            pltpu.VMEM((1, CHUNK), jnp.int32),
            pltpu.VMEM((CHUNK, feat), jnp.float32),
            pltpu.VMEM((CHUNK, feat), jnp.float32),
            pltpu.VMEM_SHARED((NPAD, feat), jnp.float32),
            pltpu.SemaphoreType.DMA,
            pltpu.SemaphoreType.DMA,
            pltpu.SemaphoreType.DMA,
            pltpu.SemaphoreType.DMA,
        ],
    )


def _tc_a_body(x_ref, w1_ref, d0_ref, d1_ref, g1_ref, dinv_ref):
    dinv = lax.rsqrt(d0_ref[...] + d1_ref[...] + 1.0)
    h = jnp.dot(x_ref[...], w1_ref[...], preferred_element_type=jnp.float32)
    g1_ref[...] = h * dinv
    dinv_ref[...] = dinv


def _tc_a(x, w1, d0, d1):
    return pl.pallas_call(
        _tc_a_body,
        grid=(NBLK,),
        in_specs=[
            pl.BlockSpec((BLK, 128), lambda i: (i, 0)),
            pl.BlockSpec((128, 32), lambda i: (0, 0)),
            pl.BlockSpec((BLK, 1), lambda i: (i, 0)),
            pl.BlockSpec((BLK, 1), lambda i: (i, 0)),
        ],
        out_specs=[
            pl.BlockSpec((BLK, 32), lambda i: (i, 0)),
            pl.BlockSpec((BLK, 1), lambda i: (i, 0)),
        ],
        out_shape=[
            jax.ShapeDtypeStruct((N_NODES, 32), jnp.float32),
            jax.ShapeDtypeStruct((N_NODES, 1), jnp.float32),
        ],
    )(x, w1, d0, d1)


def _tc_b_body(a0_ref, a1_ref, g1_ref, dinv_ref, b1_ref, w2_ref, g2_ref):
    dinv = dinv_ref[...]
    o1 = ((a0_ref[...] + a1_ref[...] + g1_ref[...]) * dinv
          + b1_ref[...])
    o1 = jnp.maximum(o1, 0.0)
    h2 = jnp.dot(o1, w2_ref[...], preferred_element_type=jnp.float32)
    g2_ref[...] = h2 * dinv


def _tc_b(a0, a1, g1, dinv, b1, w2):
    return pl.pallas_call(
        _tc_b_body,
        grid=(NBLK,),
        in_specs=[
            pl.BlockSpec((BLK, 32), lambda i: (i, 0)),
            pl.BlockSpec((BLK, 32), lambda i: (i, 0)),
            pl.BlockSpec((BLK, 32), lambda i: (i, 0)),
            pl.BlockSpec((BLK, 1), lambda i: (i, 0)),
            pl.BlockSpec((1, 32), lambda i: (0, 0)),
            pl.BlockSpec((32, 64), lambda i: (0, 0)),
        ],
        out_specs=pl.BlockSpec((BLK, 64), lambda i: (i, 0)),
        out_shape=jax.ShapeDtypeStruct((N_NODES, 64), jnp.float32),
    )(a0, a1, g1, dinv, b1, w2)


def _tc_c_body(a0_ref, a1_ref, g2_ref, dinv_ref, b2_ref, out_ref):
    out_ref[...] = ((a0_ref[...] + a1_ref[...] + g2_ref[...])
                    * dinv_ref[...] + b2_ref[...])


def _tc_c(a0, a1, g2, dinv, b2):
    return pl.pallas_call(
        _tc_c_body,
        grid=(NBLK,),
        in_specs=[
            pl.BlockSpec((BLK, 64), lambda i: (i, 0)),
            pl.BlockSpec((BLK, 64), lambda i: (i, 0)),
            pl.BlockSpec((BLK, 64), lambda i: (i, 0)),
            pl.BlockSpec((BLK, 1), lambda i: (i, 0)),
            pl.BlockSpec((1, 64), lambda i: (0, 0)),
        ],
        out_specs=pl.BlockSpec((BLK, 64), lambda i: (i, 0)),
        out_shape=jax.ShapeDtypeStruct((N_NODES, 64), jnp.float32),
    )(a0, a1, g2, dinv, b2)


@jax.jit
def kernel(x, edge_index, W1, b1, W2, b2):
    src3 = edge_index[0].astype(jnp.int32).reshape(NW, NCHUNK, CHUNK)
    dst_i32 = edge_index[1].astype(jnp.int32)
    dst3 = dst_i32.reshape(NW, NCHUNK, CHUNK)
    dst4 = dst_i32.reshape(NW, NCHUNK, 1, CHUNK)

    zeros_n = jnp.zeros((NPAD,), jnp.float32)
    zeros32 = jnp.zeros((NPAD, 32), jnp.float32)
    zeros64 = jnp.zeros((NPAD, 64), jnp.float32)

    deg = _make_deg_kernel()(dst3, zeros_n)
    d0 = deg[0, :N_NODES].reshape(N_NODES, 1)
    d1 = deg[1, :N_NODES].reshape(N_NODES, 1)

    g1, dinv = _tc_a(x, W1, d0, d1)

    agg1 = _make_agg_kernel(32)(src3, dst4, g1, zeros32)
    g2 = _tc_b(agg1[0], agg1[1], g1, dinv, b1.reshape(1, 32), W2)

    agg2 = _make_agg_kernel(64)(src3, dst4, g2, zeros64)
    return _tc_c(agg2[0], agg2[1], g2, dinv, b2.reshape(1, 64))
